# baseline (speedup 1.0000x reference)
import jax
import jax.numpy as jnp
from jax.experimental import pallas as pl
from jax.experimental.pallas import tpu as pltpu


def _pass1_body(x_ref, w1_ref, adj_ref, w2_ref, s2_ref, adj8_ref, s1_scr):
    @pl.when(pl.program_id(0) == 0)
    def _():
        s1_scr[...] = jnp.dot(
            x_ref[...].astype(jnp.bfloat16),
            w1_ref[...].astype(jnp.bfloat16),
            preferred_element_type=jnp.float32,
        ).astype(jnp.bfloat16)

    a16 = adj_ref[...].astype(jnp.bfloat16)
    adj8_ref[...] = a16.astype(jnp.float8_e4m3fn)
    b = jnp.dot(
        a16,
        s1_scr[...],
        preferred_element_type=jnp.float32,
    )
    h = jnp.maximum(b, 0.0).astype(jnp.bfloat16)
    s2_ref[...] = (
        jnp.dot(h, w2_ref[...], preferred_element_type=jnp.float32) * 0.125
    ).astype(jnp.float8_e4m3fn)


def _pass2_body(adj8_ref, s2_ref, o_ref):
    logits = 8.0 * jnp.dot(
        adj8_ref[...],
        s2_ref[...],
        preferred_element_type=jnp.float32,
    )
    m = jnp.max(logits, axis=1, keepdims=True)
    lse = jnp.log(jnp.sum(jnp.exp(logits - m), axis=1, keepdims=True)) + m
    o_ref[...] = logits - lse


def kernel(adj, x, W1, W2):
    N, D = x.shape
    H = W1.shape[1]
    C = W2.shape[1]
    BM = 400
    BM2 = 2000
    assert N % BM == 0 and N % BM2 == 0

    s2, adj8 = pl.pallas_call(
        _pass1_body,
        grid=(N // BM,),
        in_specs=[
            pl.BlockSpec((N, D), lambda i: (0, 0)),
            pl.BlockSpec((D, H), lambda i: (0, 0)),
            pl.BlockSpec((BM, N), lambda i: (i, 0)),
            pl.BlockSpec((H, C), lambda i: (0, 0)),
        ],
        out_specs=[
            pl.BlockSpec((BM, C), lambda i: (i, 0)),
            pl.BlockSpec((BM, N), lambda i: (i, 0)),
        ],
        out_shape=[
            jax.ShapeDtypeStruct((N, C), jnp.float8_e4m3fn),
            jax.ShapeDtypeStruct((N, N), jnp.float8_e4m3fn),
        ],
        scratch_shapes=[pltpu.VMEM((N, H), jnp.bfloat16)],
    )(x, W1, adj, W2.astype(jnp.bfloat16))

    out = pl.pallas_call(
        _pass2_body,
        grid=(N // BM2,),
        in_specs=[
            pl.BlockSpec((BM2, N), lambda i: (i, 0)),
            pl.BlockSpec((N, C), lambda i: (0, 0)),
        ],
        out_specs=pl.BlockSpec((BM2, C), lambda i: (i, 0)),
        out_shape=jax.ShapeDtypeStruct((N, C), jnp.float32),
        compiler_params=pltpu.CompilerParams(
            vmem_limit_bytes=64 * 1024 * 1024,
        ),
    )(adj8, s2)
    return out


# speedup vs baseline: 1.0223x; 1.0223x over previous
import jax
import jax.numpy as jnp
from jax.experimental import pallas as pl
from jax.experimental.pallas import tpu as pltpu


def _pass1_body(x_ref, w1_ref, adj_ref, w2_ref, s2_ref, adj8_ref, s1_scr):
    @pl.when(pl.program_id(0) == 0)
    def _():
        s1_scr[...] = jnp.dot(
            x_ref[...].astype(jnp.bfloat16),
            w1_ref[...].astype(jnp.bfloat16),
            preferred_element_type=jnp.float32,
        ).astype(jnp.bfloat16)

    a16 = adj_ref[...].astype(jnp.bfloat16)
    adj8_ref[...] = a16.astype(jnp.float8_e4m3fn)
    b = jnp.dot(
        a16,
        s1_scr[...],
        preferred_element_type=jnp.float32,
    )
    h = jnp.maximum(b, 0.0).astype(jnp.bfloat16)
    s2_ref[...] = (
        jnp.dot(h, w2_ref[...], preferred_element_type=jnp.float32) * 0.125
    ).astype(jnp.float8_e4m3fn)


def _pass2_body(adj8_ref, s2_ref, o_ref):
    logits = 8.0 * jnp.dot(
        adj8_ref[...],
        s2_ref[...],
        preferred_element_type=jnp.float32,
    )
    m = jnp.max(logits, axis=1, keepdims=True)
    lse = jnp.log(jnp.sum(jnp.exp(logits - m), axis=1, keepdims=True)) + m
    o_ref[...] = logits - lse


def kernel(adj, x, W1, W2):
    N, D = x.shape
    H = W1.shape[1]
    C = W2.shape[1]
    BM = 400
    BM2 = 1000
    assert N % BM == 0 and N % BM2 == 0

    s2, adj8 = pl.pallas_call(
        _pass1_body,
        grid=(N // BM,),
        in_specs=[
            pl.BlockSpec((N, D), lambda i: (0, 0)),
            pl.BlockSpec((D, H), lambda i: (0, 0)),
            pl.BlockSpec((BM, N), lambda i: (i, 0)),
            pl.BlockSpec((H, C), lambda i: (0, 0)),
        ],
        out_specs=[
            pl.BlockSpec((BM, C), lambda i: (i, 0)),
            pl.BlockSpec((BM, N), lambda i: (i, 0)),
        ],
        out_shape=[
            jax.ShapeDtypeStruct((N, C), jnp.float8_e4m3fn),
            jax.ShapeDtypeStruct((N, N), jnp.float8_e4m3fn),
        ],
        scratch_shapes=[pltpu.VMEM((N, H), jnp.bfloat16)],
    )(x, W1, adj, W2.astype(jnp.bfloat16))

    out = pl.pallas_call(
        _pass2_body,
        grid=(N // BM2,),
        in_specs=[
            pl.BlockSpec((BM2, N), lambda i: (i, 0)),
            pl.BlockSpec((N, C), lambda i: (0, 0)),
        ],
        out_specs=pl.BlockSpec((BM2, C), lambda i: (i, 0)),
        out_shape=jax.ShapeDtypeStruct((N, C), jnp.float32),
    )(adj8, s2)
    return out
